# Initial kernel scaffold; baseline (speedup 1.0000x reference)
import jax
import jax.numpy as jnp
from jax.experimental import pallas as pl


def kernel(X, edge_index, edge_weight, W_red, b_red, W_g1, b_g1, W_g2, b_g2, W_cls, b_cls):
    raise NotImplementedError("write your pallas kernel here")


# speedup vs baseline: 1.8444x; 1.8444x over previous
import jax
import jax.numpy as jnp
from jax import lax
from jax.experimental import pallas as pl
from jax.experimental.pallas import tpu as pltpu
from jax.experimental.pallas import tpu_sc as plsc

N_TOT = 10000
N_OUT = 8000
N_EDGE = 320000
D = 128

NUM_SC = 2
NUM_TILES = 16
EPT = N_EDGE // NUM_TILES
CHUNK = 80
NCHUNK = EPT // CHUNK
LANES = 16

_f32 = jnp.float32


def _dot(a, b):
    return jnp.dot(a, b, preferred_element_type=_f32)


def _tc1_body(x_ref, wred_ref, bred_ref, wg1_ref, ei_ref,
              emb0_ref, s1_ref, src_ref, dst_ref):
    emb0 = _dot(x_ref[...], wred_ref[...]) + bred_ref[0]
    emb0_ref[...] = emb0
    s1_ref[...] = _dot(emb0, wg1_ref[...])
    src_ref[...] = ei_ref[0]
    dst_ref[...] = ei_ref[1]


def _tc2_body(agg_ref, emb0_ref, bg1_ref, wg2_ref, acc01_ref, s2_ref):
    emb1 = jnp.maximum(agg_ref[...] + bg1_ref[0], 0.0)
    acc01_ref[...] = emb0_ref[...] + emb1
    s2_ref[...] = _dot(emb1, wg2_ref[...])


def _tc3_body(acc01_ref, agg2_ref, bg2_ref, wcls_ref, bcls_ref, out_ref):
    emb2 = jnp.maximum(agg2_ref[...] + bg2_ref[0], 0.0)
    out = (acc01_ref[...] + emb2) * (1.0 / 3.0)
    z = _dot(out, wcls_ref[...]) + bcls_ref[0]
    m = jnp.max(z, axis=-1, keepdims=True)
    ez = jnp.exp(z - m)
    lse = jnp.log(jnp.sum(ez, axis=-1, keepdims=True)) + m
    out_ref[...] = z - lse


def _tc1(X, W_red, b_red, W_g1, edge_index):
    return pl.pallas_call(
        _tc1_body,
        out_shape=[jax.ShapeDtypeStruct((N_TOT, D), _f32),
                   jax.ShapeDtypeStruct((N_TOT, D), _f32),
                   jax.ShapeDtypeStruct((N_EDGE,), jnp.int32),
                   jax.ShapeDtypeStruct((N_EDGE,), jnp.int32)],
    )(X, W_red, b_red.reshape(1, D), W_g1, edge_index)


def _tc2(agg1, emb0, b_g1, W_g2):
    return pl.pallas_call(
        _tc2_body,
        out_shape=[jax.ShapeDtypeStruct((N_TOT, D), _f32),
                   jax.ShapeDtypeStruct((N_TOT, D), _f32)],
    )(agg1, emb0, b_g1.reshape(1, D), W_g2)


def _tc3(acc01, agg2, b_g2, W_cls, b_cls):
    node_spec = pl.BlockSpec((N_OUT, D), lambda i: (0, 0))
    return pl.pallas_call(
        _tc3_body,
        grid=(1,),
        in_specs=[node_spec, node_spec,
                  pl.BlockSpec((1, D), lambda i: (0, 0)),
                  pl.BlockSpec((D, D), lambda i: (0, 0)),
                  pl.BlockSpec((1, D), lambda i: (0, 0))],
        out_specs=pl.BlockSpec((N_OUT, D), lambda i: (0, 0)),
        out_shape=jax.ShapeDtypeStruct((N_OUT, D), _f32),
    )(acc01, agg2, b_g2.reshape(1, D), W_cls, b_cls.reshape(1, D))


def _bcast_lane(vec, lane):
    return lax.gather(
        vec,
        jnp.full((LANES, 1), lane, jnp.int32),
        lax.GatherDimensionNumbers(
            offset_dims=(), collapsed_slice_dims=(0,), start_index_map=(0,)),
        (1,),
        mode=lax.GatherScatterMode.PROMISE_IN_BOUNDS,
    )


MAIN_PER_SC = 4608
MAIN_COVER = NUM_SC * MAIN_PER_SC
TAIL_PER_SC = (N_TOT - MAIN_COVER) // NUM_SC
MAIN_TILE = MAIN_PER_SC // NUM_TILES
TAIL_TILE = (TAIL_PER_SC // NUM_TILES) // 8 * 8
TAIL_REM = TAIL_PER_SC - NUM_TILES * TAIL_TILE


def _sc_spmm_body(sup_hbm, src_hbm, dst_hbm, w_hbm, out_hbm,
                  src_v, dst_v, w_v, dstc_v, rows_v, acc, sem):
    c = lax.axis_index("c")
    s = lax.axis_index("s")

    ebase = s * EPT
    pltpu.sync_copy(src_hbm.at[pl.ds(ebase, EPT)], src_v)
    pltpu.sync_copy(dst_hbm.at[pl.ds(ebase, EPT)], dst_v)
    pltpu.sync_copy(w_hbm.at[pl.ds(ebase, EPT)], w_v)

    zero = jnp.zeros((LANES,), _f32)
    for r in range(CHUNK):
        for f in range(D // LANES):
            rows_v[r, f * LANES:(f + 1) * LANES] = zero

    def zero_acc(start, nrows):
        nfull = nrows // CHUNK
        for k in range(nfull):
            pltpu.sync_copy(rows_v, acc.at[pl.ds(start + k * CHUNK, CHUNK)])
        rem = nrows - nfull * CHUNK
        if rem:
            pltpu.sync_copy(rows_v.at[pl.ds(0, rem)],
                            acc.at[pl.ds(start + nfull * CHUNK, rem)])

    def run_pass(lo, nrows):
        def chunk_body(j, carry):
            eoff = j * CHUNK
            pltpu.async_copy(
                sup_hbm.at[src_v.at[pl.ds(eoff, CHUNK)]], rows_v, sem).wait()
            for g in range(CHUNK // LANES):
                sl = pl.ds(eoff + g * LANES, LANES)
                d = dst_v[sl] - lo
                own = (d >= 0) & (d < nrows)
                dstc_v[pl.ds(g * LANES, LANES)] = jnp.where(own, d, 0)
                w_reg = jnp.where(own, w_v[sl], 0.0)
                for lane in range(LANES):
                    e = g * LANES + lane
                    bw = _bcast_lane(w_reg, lane)
                    for f in range(D // LANES):
                        fs = pl.ds(f * LANES, LANES)
                        rows_v[e, fs] = rows_v[e, fs] * bw
            pltpu.sync_copy(rows_v, acc.at[dstc_v], add=True)
            return carry
        lax.fori_loop(0, NCHUNK, chunk_body, 0)

    zero_acc(s * MAIN_TILE, MAIN_TILE)
    plsc.subcore_barrier()
    run_pass(c * MAIN_PER_SC, MAIN_PER_SC)
    plsc.subcore_barrier()
    pltpu.sync_copy(
        acc.at[pl.ds(s * MAIN_TILE, MAIN_TILE)],
        out_hbm.at[pl.ds(c * MAIN_PER_SC + s * MAIN_TILE, MAIN_TILE)])
    plsc.subcore_barrier()

    zero_acc(s * TAIL_TILE, TAIL_TILE)
    @pl.when(s == 0)
    def _zero_tail():
        zero_acc(NUM_TILES * TAIL_TILE, TAIL_REM)
    plsc.subcore_barrier()
    lo_b = MAIN_COVER + c * TAIL_PER_SC
    run_pass(lo_b, TAIL_PER_SC)
    plsc.subcore_barrier()
    pltpu.sync_copy(acc.at[pl.ds(s * TAIL_TILE, TAIL_TILE)],
                    out_hbm.at[pl.ds(lo_b + s * TAIL_TILE, TAIL_TILE)])
    @pl.when(s == 0)
    def _write_tail():
        pltpu.sync_copy(
            acc.at[pl.ds(NUM_TILES * TAIL_TILE, TAIL_REM)],
            out_hbm.at[pl.ds(lo_b + NUM_TILES * TAIL_TILE, TAIL_REM)])


def _sc_spmm(sup, src_t, dst_t, w_t):
    mesh = plsc.VectorSubcoreMesh(core_axis_name="c", subcore_axis_name="s")
    f = pl.kernel(
        _sc_spmm_body,
        mesh=mesh,
        out_type=jax.ShapeDtypeStruct((N_TOT, D), _f32),
        scratch_types=[
            pltpu.VMEM((EPT,), jnp.int32),
            pltpu.VMEM((EPT,), jnp.int32),
            pltpu.VMEM((EPT,), _f32),
            pltpu.VMEM((CHUNK,), jnp.int32),
            pltpu.VMEM((CHUNK, D), _f32),
            pltpu.VMEM_SHARED((MAIN_PER_SC, D), _f32),
            pltpu.SemaphoreType.DMA,
        ],
    )
    return f(sup, src_t, dst_t, w_t)


def kernel(X, edge_index, edge_weight, W_red, b_red, W_g1, b_g1,
           W_g2, b_g2, W_cls, b_cls):
    emb0, s1, src_f, dst_f = _tc1(X, W_red, b_red, W_g1, edge_index)
    agg1 = _sc_spmm(s1, src_f, dst_f, edge_weight)
    acc01, s2 = _tc2(agg1, emb0, b_g1, W_g2)
    agg2 = _sc_spmm(s2, src_f, dst_f, edge_weight)
    return _tc3(acc01, agg2, b_g2, W_cls, b_cls)


# speedup vs baseline: 2.6561x; 1.4401x over previous
import jax
import jax.numpy as jnp
from jax import lax
from jax.experimental import pallas as pl
from jax.experimental.pallas import tpu as pltpu
from jax.experimental.pallas import tpu_sc as plsc

N_TOT = 10000
N_OUT = 8000
N_EDGE = 320000
D = 128

NUM_SC = 2
NUM_TILES = 16
EPT = N_EDGE // NUM_TILES
CHUNK = 80
NCHUNK = EPT // CHUNK
LANES = 16

_f32 = jnp.float32


def _dot(a, b):
    return jnp.dot(a, b, preferred_element_type=_f32)


def _tc1_body(x_ref, wred_ref, bred_ref, wg1_ref, ei_ref,
              emb0_ref, s1_ref, src_ref, dst_ref):
    emb0 = _dot(x_ref[...], wred_ref[...]) + bred_ref[0]
    emb0_ref[...] = emb0
    s1_ref[...] = _dot(emb0, wg1_ref[...])
    src_ref[...] = ei_ref[0]
    dst_ref[...] = ei_ref[1]


def _tc2_body(agg_ref, emb0_ref, bg1_ref, wg2_ref, acc01_ref, s2_ref):
    emb1 = jnp.maximum(agg_ref[...] + bg1_ref[0], 0.0)
    acc01_ref[...] = emb0_ref[...] + emb1
    s2_ref[...] = _dot(emb1, wg2_ref[...])


def _tc3_body(acc01_ref, agg2_ref, bg2_ref, wcls_ref, bcls_ref, out_ref):
    emb2 = jnp.maximum(agg2_ref[...] + bg2_ref[0], 0.0)
    out = (acc01_ref[...] + emb2) * (1.0 / 3.0)
    z = _dot(out, wcls_ref[...]) + bcls_ref[0]
    m = jnp.max(z, axis=-1, keepdims=True)
    ez = jnp.exp(z - m)
    lse = jnp.log(jnp.sum(ez, axis=-1, keepdims=True)) + m
    out_ref[...] = z - lse


def _tc1(X, W_red, b_red, W_g1, edge_index):
    return pl.pallas_call(
        _tc1_body,
        out_shape=[jax.ShapeDtypeStruct((N_TOT, D), _f32),
                   jax.ShapeDtypeStruct((N_TOT, D), _f32),
                   jax.ShapeDtypeStruct((N_EDGE,), jnp.int32),
                   jax.ShapeDtypeStruct((N_EDGE,), jnp.int32)],
    )(X, W_red, b_red.reshape(1, D), W_g1, edge_index)


def _tc2(agg1, emb0, b_g1, W_g2):
    return pl.pallas_call(
        _tc2_body,
        out_shape=[jax.ShapeDtypeStruct((N_TOT, D), _f32),
                   jax.ShapeDtypeStruct((N_TOT, D), _f32)],
    )(agg1, emb0, b_g1.reshape(1, D), W_g2)


def _tc3(acc01, agg2, b_g2, W_cls, b_cls):
    node_spec = pl.BlockSpec((N_OUT, D), lambda i: (0, 0))
    return pl.pallas_call(
        _tc3_body,
        grid=(1,),
        in_specs=[node_spec, node_spec,
                  pl.BlockSpec((1, D), lambda i: (0, 0)),
                  pl.BlockSpec((D, D), lambda i: (0, 0)),
                  pl.BlockSpec((1, D), lambda i: (0, 0))],
        out_specs=pl.BlockSpec((N_OUT, D), lambda i: (0, 0)),
        out_shape=jax.ShapeDtypeStruct((N_OUT, D), _f32),
    )(acc01, agg2, b_g2.reshape(1, D), W_cls, b_cls.reshape(1, D))


def _bcast_lane(vec, lane):
    return lax.gather(
        vec,
        jnp.full((LANES, 1), lane, jnp.int32),
        lax.GatherDimensionNumbers(
            offset_dims=(), collapsed_slice_dims=(0,), start_index_map=(0,)),
        (1,),
        mode=lax.GatherScatterMode.PROMISE_IN_BOUNDS,
    )


MAIN_PER_SC = 4608
MAIN_COVER = NUM_SC * MAIN_PER_SC
TAIL_PER_SC = (N_TOT - MAIN_COVER) // NUM_SC
MAIN_TILE = MAIN_PER_SC // NUM_TILES
TAIL_TILE = (TAIL_PER_SC // NUM_TILES) // 8 * 8
TAIL_REM = TAIL_PER_SC - NUM_TILES * TAIL_TILE


def _sc_spmm_body(sup_hbm, src_hbm, dst_hbm, w_hbm, out_hbm,
                  src_v, dst_v, w_v, dstc0, dstc1, rows0, rows1, zbuf, acc,
                  gsem0, gsem1):
    c = lax.axis_index("c")
    s = lax.axis_index("s")
    rows_b = (rows0, rows1)
    dstc_b = (dstc0, dstc1)
    gsem_b = (gsem0, gsem1)

    ebase = s * EPT
    pltpu.sync_copy(src_hbm.at[pl.ds(ebase, EPT)], src_v)
    pltpu.sync_copy(dst_hbm.at[pl.ds(ebase, EPT)], dst_v)
    pltpu.sync_copy(w_hbm.at[pl.ds(ebase, EPT)], w_v)

    zero = jnp.zeros((LANES,), _f32)
    for r in range(CHUNK):
        for f in range(D // LANES):
            zbuf[r, f * LANES:(f + 1) * LANES] = zero

    def zero_acc(start, nrows):
        nfull = nrows // CHUNK
        for k in range(nfull):
            pltpu.sync_copy(zbuf, acc.at[pl.ds(start + k * CHUNK, CHUNK)])
        rem = nrows - nfull * CHUNK
        if rem:
            pltpu.sync_copy(zbuf.at[pl.ds(0, rem)],
                            acc.at[pl.ds(start + nfull * CHUNK, rem)])

    def gissue(j, b):
        pltpu.async_copy(sup_hbm.at[src_v.at[pl.ds(j * CHUNK, CHUNK)]],
                         rows_b[b], gsem_b[b])

    def run_pass(lo, nrows):
        gissue(0, 0)
        gissue(1, 1)

        def process(j, b):
            rv = rows_b[b]
            dcv = dstc_b[b]
            pltpu.make_async_copy(
                sup_hbm.at[src_v.at[pl.ds(0, CHUNK)]], rv, gsem_b[b]).wait()

            def group(g, carry):
                sl = pl.ds(j * CHUNK + g * LANES, LANES)
                d = dst_v[sl] - lo
                own = (d >= 0) & (d < nrows)
                dcv[pl.ds(g * LANES, LANES)] = jnp.where(own, d, 0)
                w_reg = jnp.where(own, w_v[sl], 0.0)
                for lane in range(LANES):
                    bw = _bcast_lane(w_reg, lane)
                    e = g * LANES + lane
                    for f in range(D // LANES):
                        fs = pl.ds(f * LANES, LANES)
                        rv[e, fs] = rv[e, fs] * bw
                return carry
            lax.fori_loop(0, CHUNK // LANES, group, 0)
            pltpu.sync_copy(rv, acc.at[dcv], add=True)
            @pl.when(j + 2 < NCHUNK)
            def _prefetch():
                gissue(j + 2, b)

        def pair_body(jj, carry):
            process(jj * 2, 0)
            process(jj * 2 + 1, 1)
            return carry
        lax.fori_loop(0, NCHUNK // 2, pair_body, 0)

    zero_acc(s * MAIN_TILE, MAIN_TILE)
    plsc.subcore_barrier()
    run_pass(c * MAIN_PER_SC, MAIN_PER_SC)
    plsc.subcore_barrier()
    pltpu.sync_copy(
        acc.at[pl.ds(s * MAIN_TILE, MAIN_TILE)],
        out_hbm.at[pl.ds(c * MAIN_PER_SC + s * MAIN_TILE, MAIN_TILE)])
    plsc.subcore_barrier()

    zero_acc(s * TAIL_TILE, TAIL_TILE)
    @pl.when(s == 0)
    def _zero_tail():
        zero_acc(NUM_TILES * TAIL_TILE, TAIL_REM)
    plsc.subcore_barrier()
    lo_b = MAIN_COVER + c * TAIL_PER_SC
    run_pass(lo_b, TAIL_PER_SC)
    plsc.subcore_barrier()
    pltpu.sync_copy(acc.at[pl.ds(s * TAIL_TILE, TAIL_TILE)],
                    out_hbm.at[pl.ds(lo_b + s * TAIL_TILE, TAIL_TILE)])
    @pl.when(s == 0)
    def _write_tail():
        pltpu.sync_copy(
            acc.at[pl.ds(NUM_TILES * TAIL_TILE, TAIL_REM)],
            out_hbm.at[pl.ds(lo_b + NUM_TILES * TAIL_TILE, TAIL_REM)])


def _sc_spmm(sup, src_t, dst_t, w_t):
    mesh = plsc.VectorSubcoreMesh(core_axis_name="c", subcore_axis_name="s")
    f = pl.kernel(
        _sc_spmm_body,
        mesh=mesh,
        out_type=jax.ShapeDtypeStruct((N_TOT, D), _f32),
        scratch_types=[
            pltpu.VMEM((EPT,), jnp.int32),
            pltpu.VMEM((EPT,), jnp.int32),
            pltpu.VMEM((EPT,), _f32),
            pltpu.VMEM((CHUNK,), jnp.int32),
            pltpu.VMEM((CHUNK,), jnp.int32),
            pltpu.VMEM((CHUNK, D), _f32),
            pltpu.VMEM((CHUNK, D), _f32),
            pltpu.VMEM((CHUNK, D), _f32),
            pltpu.VMEM_SHARED((MAIN_PER_SC, D), _f32),
            pltpu.SemaphoreType.DMA,
            pltpu.SemaphoreType.DMA,
        ],
    )
    return f(sup, src_t, dst_t, w_t)


def kernel(X, edge_index, edge_weight, W_red, b_red, W_g1, b_g1,
           W_g2, b_g2, W_cls, b_cls):
    emb0, s1, src_f, dst_f = _tc1(X, W_red, b_red, W_g1, edge_index)
    agg1 = _sc_spmm(s1, src_f, dst_f, edge_weight)
    acc01, s2 = _tc2(agg1, emb0, b_g1, W_g2)
    agg2 = _sc_spmm(s2, src_f, dst_f, edge_weight)
    return _tc3(acc01, agg2, b_g2, W_cls, b_cls)
